# baseline (speedup 1.0000x reference)
import functools

import jax
import jax.numpy as jnp
from jax import lax
from jax.experimental import pallas as pl
from jax.experimental.pallas import tpu as pltpu
from jax.experimental.pallas import tpu_sc as plsc

N_NEIGHBOURS = 5
TQ = 256
BK = 1024
NB = 98
K_PAD = 100352
NQ = 4096
PAD_VAL = 1e4
BIGF = 3e38

SC_NC = 2
SC_NS = 16
SC_NW = SC_NC * SC_NS
SC_CHUNK = 64


def _dist_kernel(q_ref, k_ref, dist_ref, bm_ref, acc_ref):
    ki = pl.program_id(1)

    @pl.when(ki == 0)
    def _init():
        acc_ref[...] = jnp.full((TQ, 128), BIGF, jnp.float32)

    q = q_ref[...]
    k = k_ref[...]
    q2 = jnp.sum(q * q, axis=1, keepdims=True)
    k2 = jnp.sum(k * k, axis=1)
    dots = jax.lax.dot_general(
        q, k, (((1,), (1,)), ((), ())),
        preferred_element_type=jnp.float32)
    dist = q2 + k2[None, :] - 2.0 * dots
    dist_ref[...] = dist
    m = jnp.min(dist, axis=1, keepdims=True)
    lane = jax.lax.broadcasted_iota(jnp.int32, (TQ, 128), 1)
    acc_ref[...] = jnp.where(lane == ki, m, acc_ref[...])

    @pl.when(ki == NB - 1)
    def _emit():
        bm_ref[...] = acc_ref[...]


def _select_kernel(bm_ref, cb_ref, fi_ref):
    qi = pl.program_id(0)
    lane = jax.lax.broadcasted_iota(jnp.int32, (TQ, 128), 1)
    lane_f = lane.astype(jnp.float32)
    cv = jnp.where(lane < NB, bm_ref[...], BIGF)
    cols = []
    for _ in range(N_NEIGHBOURS):
        m = jnp.min(cv, axis=1, keepdims=True)
        key = jnp.where(cv == m, lane_f, BIGF)
        pos = jnp.min(key, axis=1, keepdims=True)
        cv = jnp.where(key == pos, BIGF, cv)
        cols.append(pos)
    for a, b in ((0, 1), (3, 4), (2, 4), (2, 3), (1, 4),
                 (0, 3), (0, 2), (1, 3), (1, 2)):
        lo = jnp.minimum(cols[a], cols[b])
        hi = jnp.maximum(cols[a], cols[b])
        cols[a], cols[b] = lo, hi
    cb = jnp.concatenate(cols, axis=1).astype(jnp.int32)
    cb_ref[...] = cb
    row = qi * TQ + jax.lax.broadcasted_iota(jnp.int32, (TQ, 1), 0)
    fi_ref[...] = cb * NQ + row


def _sc_gather(table_ref, idx_ref, out_ref, idx_v, rows_v, sem):
    b_per_w = (NQ * N_NEIGHBOURS) // SC_NW
    wid = lax.axis_index("s") * SC_NC + lax.axis_index("c")
    base = wid * b_per_w
    for c in range(b_per_w // SC_CHUNK):
        off = base + c * SC_CHUNK
        pltpu.sync_copy(idx_ref.at[pl.ds(off, SC_CHUNK)], idx_v)
        pltpu.async_copy(table_ref.at[idx_v], rows_v, sem).wait()
        pltpu.sync_copy(rows_v, out_ref.at[pl.ds(off, SC_CHUNK)])


def _final_kernel(g0, g1, g2, g3, g4, cb_ref, scores_ref, idx_ref):
    g = [r[...] for r in (g0, g1, g2, g3, g4)]
    cbf = cb_ref[...].astype(jnp.float32)
    lane_f = jax.lax.broadcasted_iota(
        jnp.int32, (TQ, BK), 1).astype(jnp.float32)
    gpos = [cbf[:, j:j + 1] * BK + lane_f for j in range(N_NEIGHBOURS)]
    vals = []
    idxs = []
    for _ in range(N_NEIGHBOURS):
        s = g[0]
        for j in range(1, N_NEIGHBOURS):
            s = jnp.minimum(s, g[j])
        m = jnp.min(s, axis=1, keepdims=True)
        keys = [jnp.where(g[j] == m, gpos[j], BIGF) for j in range(N_NEIGHBOURS)]
        ks = keys[0]
        for j in range(1, N_NEIGHBOURS):
            ks = jnp.minimum(ks, keys[j])
        pos = jnp.min(ks, axis=1, keepdims=True)
        g = [jnp.where(keys[j] == pos, BIGF, g[j]) for j in range(N_NEIGHBOURS)]
        vals.append(m)
        idxs.append(pos)
    vals5 = jnp.concatenate(vals, axis=1)
    scores_ref[...] = jnp.mean(vals5, axis=1, keepdims=True)
    idx_ref[...] = jnp.concatenate(idxs, axis=1).astype(jnp.int32)


def _gather_candidates(dist_flat, fi_flat):
    nrows = fi_flat.shape[0]
    gather = pl.kernel(
        _sc_gather,
        out_type=jax.ShapeDtypeStruct((nrows, BK), jnp.float32),
        mesh=plsc.VectorSubcoreMesh(core_axis_name="c", subcore_axis_name="s"),
        scratch_types=[
            pltpu.VMEM((SC_CHUNK,), jnp.int32),
            pltpu.VMEM((SC_CHUNK, BK), jnp.float32),
            pltpu.SemaphoreType.DMA,
        ],
    )
    return gather(dist_flat, fi_flat)


@jax.jit
def kernel(queries, keys):
    Q, D = queries.shape
    K, _ = keys.shape
    keys_p = jnp.pad(keys, ((0, K_PAD - K), (0, 0)), constant_values=PAD_VAL)

    dist, bm = pl.pallas_call(
        _dist_kernel,
        grid=(Q // TQ, NB),
        in_specs=[
            pl.BlockSpec((TQ, D), lambda qi, ki: (qi, 0)),
            pl.BlockSpec((BK, D), lambda qi, ki: (ki, 0)),
        ],
        out_specs=[
            pl.BlockSpec((TQ, BK), lambda qi, ki: (ki * (NQ // TQ) + qi, 0)),
            pl.BlockSpec((TQ, 128), lambda qi, ki: (qi, 0)),
        ],
        out_shape=[
            jax.ShapeDtypeStruct((NB * Q, BK), jnp.float32),
            jax.ShapeDtypeStruct((Q, 128), jnp.float32),
        ],
        scratch_shapes=[pltpu.VMEM((TQ, 128), jnp.float32)],
        compiler_params=pltpu.CompilerParams(
            dimension_semantics=("parallel", "arbitrary"),
        ),
    )(queries, keys_p)

    cb, fi = pl.pallas_call(
        _select_kernel,
        grid=(Q // TQ,),
        in_specs=[pl.BlockSpec((TQ, 128), lambda qi: (qi, 0))],
        out_specs=[
            pl.BlockSpec((TQ, N_NEIGHBOURS), lambda qi: (qi, 0)),
            pl.BlockSpec((TQ, N_NEIGHBOURS), lambda qi: (qi, 0)),
        ],
        out_shape=[
            jax.ShapeDtypeStruct((Q, N_NEIGHBOURS), jnp.int32),
            jax.ShapeDtypeStruct((Q, N_NEIGHBOURS), jnp.int32),
        ],
    )(bm)

    gathered = _gather_candidates(dist, fi.T.reshape(Q * N_NEIGHBOURS))

    scores2d, topk_idx = pl.pallas_call(
        _final_kernel,
        grid=(Q // TQ,),
        in_specs=[
            pl.BlockSpec((TQ, BK), lambda qi, j=j: (j * (NQ // TQ) + qi, 0))
            for j in range(N_NEIGHBOURS)
        ] + [
            pl.BlockSpec((TQ, N_NEIGHBOURS), lambda qi: (qi, 0)),
        ],
        out_specs=[
            pl.BlockSpec((TQ, 1), lambda qi: (qi, 0)),
            pl.BlockSpec((TQ, N_NEIGHBOURS), lambda qi: (qi, 0)),
        ],
        out_shape=[
            jax.ShapeDtypeStruct((Q, 1), jnp.float32),
            jax.ShapeDtypeStruct((Q, N_NEIGHBOURS), jnp.int32),
        ],
    )(gathered, gathered, gathered, gathered, gathered, cb)
    return scores2d[:, 0], topk_idx


# speedup vs baseline: 1.1036x; 1.1036x over previous
import functools

import jax
import jax.numpy as jnp
from jax import lax
from jax.experimental import pallas as pl
from jax.experimental.pallas import tpu as pltpu
from jax.experimental.pallas import tpu_sc as plsc

N_NEIGHBOURS = 5
TQ = 256
BK = 1024
NB = 98
K_PAD = 100352
NQ = 4096
PAD_VAL = 1e4
BIGF = 3e38

SC_NC = 2
SC_NS = 16
SC_NW = SC_NC * SC_NS
SC_CHUNK = 64


def _dist_kernel(q_ref, k_ref, dist_ref, bm_ref):
    ki = pl.program_id(0)
    qi = pl.program_id(1)
    rows = pl.ds(qi * TQ, TQ)

    @pl.when(ki == 0)
    def _init():
        bm_ref[rows, :] = jnp.full((TQ, 128), BIGF, jnp.float32)

    q = q_ref[...]
    k = k_ref[...]
    q2 = jnp.sum(q * q, axis=1, keepdims=True)
    k2 = jnp.sum(k * k, axis=1)
    dots = jax.lax.dot_general(
        q, k, (((1,), (1,)), ((), ())),
        preferred_element_type=jnp.float32)
    dist = q2 + k2[None, :] - 2.0 * dots
    dist_ref[...] = dist
    m = jnp.min(dist, axis=1, keepdims=True)
    lane = jax.lax.broadcasted_iota(jnp.int32, (TQ, 128), 1)
    bm_ref[rows, :] = jnp.where(lane == ki, m, bm_ref[rows, :])


def _select_kernel(bm_ref, cb_ref, fi_ref):
    qi = pl.program_id(0)
    lane = jax.lax.broadcasted_iota(jnp.int32, (TQ, 128), 1)
    lane_f = lane.astype(jnp.float32)
    cv = jnp.where(lane < NB, bm_ref[...], BIGF)
    cols = []
    for _ in range(N_NEIGHBOURS):
        m = jnp.min(cv, axis=1, keepdims=True)
        key = jnp.where(cv == m, lane_f, BIGF)
        pos = jnp.min(key, axis=1, keepdims=True)
        cv = jnp.where(key == pos, BIGF, cv)
        cols.append(pos)
    for a, b in ((0, 1), (3, 4), (2, 4), (2, 3), (1, 4),
                 (0, 3), (0, 2), (1, 3), (1, 2)):
        lo = jnp.minimum(cols[a], cols[b])
        hi = jnp.maximum(cols[a], cols[b])
        cols[a], cols[b] = lo, hi
    cb = jnp.concatenate(cols, axis=1).astype(jnp.int32)
    cb_ref[...] = cb
    row = qi * TQ + jax.lax.broadcasted_iota(jnp.int32, (TQ, 1), 0)
    fi_ref[...] = cb * NQ + row


def _sc_gather(table_ref, idx_ref, out_ref, idx_v, rows_v, sem):
    b_per_w = (NQ * N_NEIGHBOURS) // SC_NW
    wid = lax.axis_index("s") * SC_NC + lax.axis_index("c")
    base = wid * b_per_w
    for c in range(b_per_w // SC_CHUNK):
        off = base + c * SC_CHUNK
        pltpu.sync_copy(idx_ref.at[pl.ds(off, SC_CHUNK)], idx_v)
        pltpu.async_copy(table_ref.at[idx_v], rows_v, sem).wait()
        pltpu.sync_copy(rows_v, out_ref.at[pl.ds(off, SC_CHUNK)])


def _final_kernel(g0, g1, g2, g3, g4, cb_ref, scores_ref, idx_ref):
    g = [r[...] for r in (g0, g1, g2, g3, g4)]
    cbf = cb_ref[...].astype(jnp.float32)
    lane_f = jax.lax.broadcasted_iota(
        jnp.int32, (TQ, BK), 1).astype(jnp.float32)
    gpos = [cbf[:, j:j + 1] * BK + lane_f for j in range(N_NEIGHBOURS)]
    vals = []
    idxs = []
    for _ in range(N_NEIGHBOURS):
        s = g[0]
        for j in range(1, N_NEIGHBOURS):
            s = jnp.minimum(s, g[j])
        m = jnp.min(s, axis=1, keepdims=True)
        keys = [jnp.where(g[j] == m, gpos[j], BIGF) for j in range(N_NEIGHBOURS)]
        ks = keys[0]
        for j in range(1, N_NEIGHBOURS):
            ks = jnp.minimum(ks, keys[j])
        pos = jnp.min(ks, axis=1, keepdims=True)
        g = [jnp.where(keys[j] == pos, BIGF, g[j]) for j in range(N_NEIGHBOURS)]
        vals.append(m)
        idxs.append(pos)
    vals5 = jnp.concatenate(vals, axis=1)
    scores_ref[...] = jnp.mean(vals5, axis=1, keepdims=True)
    idx_ref[...] = jnp.concatenate(idxs, axis=1).astype(jnp.int32)


def _gather_candidates(dist_flat, fi_flat):
    nrows = fi_flat.shape[0]
    gather = pl.kernel(
        _sc_gather,
        out_type=jax.ShapeDtypeStruct((nrows, BK), jnp.float32),
        mesh=plsc.VectorSubcoreMesh(core_axis_name="c", subcore_axis_name="s"),
        scratch_types=[
            pltpu.VMEM((SC_CHUNK,), jnp.int32),
            pltpu.VMEM((SC_CHUNK, BK), jnp.float32),
            pltpu.SemaphoreType.DMA,
        ],
    )
    return gather(dist_flat, fi_flat)


@jax.jit
def kernel(queries, keys):
    Q, D = queries.shape
    K, _ = keys.shape
    keys_p = jnp.pad(keys, ((0, K_PAD - K), (0, 0)), constant_values=PAD_VAL)

    dist, bm = pl.pallas_call(
        _dist_kernel,
        grid=(NB, Q // TQ),
        in_specs=[
            pl.BlockSpec((TQ, D), lambda ki, qi: (qi, 0)),
            pl.BlockSpec((BK, D), lambda ki, qi: (ki, 0)),
        ],
        out_specs=[
            pl.BlockSpec((TQ, BK), lambda ki, qi: (ki * (NQ // TQ) + qi, 0)),
            pl.BlockSpec((NQ, 128), lambda ki, qi: (0, 0)),
        ],
        out_shape=[
            jax.ShapeDtypeStruct((NB * Q, BK), jnp.float32),
            jax.ShapeDtypeStruct((Q, 128), jnp.float32),
        ],
        compiler_params=pltpu.CompilerParams(
            dimension_semantics=("arbitrary", "arbitrary"),
        ),
    )(queries, keys_p)

    cb, fi = pl.pallas_call(
        _select_kernel,
        grid=(Q // TQ,),
        in_specs=[pl.BlockSpec((TQ, 128), lambda qi: (qi, 0))],
        out_specs=[
            pl.BlockSpec((TQ, N_NEIGHBOURS), lambda qi: (qi, 0)),
            pl.BlockSpec((TQ, N_NEIGHBOURS), lambda qi: (qi, 0)),
        ],
        out_shape=[
            jax.ShapeDtypeStruct((Q, N_NEIGHBOURS), jnp.int32),
            jax.ShapeDtypeStruct((Q, N_NEIGHBOURS), jnp.int32),
        ],
    )(bm)

    gathered = _gather_candidates(dist, fi.T.reshape(Q * N_NEIGHBOURS))

    scores2d, topk_idx = pl.pallas_call(
        _final_kernel,
        grid=(Q // TQ,),
        in_specs=[
            pl.BlockSpec((TQ, BK), lambda qi, j=j: (j * (NQ // TQ) + qi, 0))
            for j in range(N_NEIGHBOURS)
        ] + [
            pl.BlockSpec((TQ, N_NEIGHBOURS), lambda qi: (qi, 0)),
        ],
        out_specs=[
            pl.BlockSpec((TQ, 1), lambda qi: (qi, 0)),
            pl.BlockSpec((TQ, N_NEIGHBOURS), lambda qi: (qi, 0)),
        ],
        out_shape=[
            jax.ShapeDtypeStruct((Q, 1), jnp.float32),
            jax.ShapeDtypeStruct((Q, N_NEIGHBOURS), jnp.int32),
        ],
    )(gathered, gathered, gathered, gathered, gathered, cb)
    return scores2d[:, 0], topk_idx


# speedup vs baseline: 1.5168x; 1.3744x over previous
import functools

import jax
import jax.numpy as jnp
from jax import lax
from jax.experimental import pallas as pl
from jax.experimental.pallas import tpu as pltpu
from jax.experimental.pallas import tpu_sc as plsc

N_NEIGHBOURS = 5
TQ = 256
TQD = 512
BK = 1024
NB = 98
K_PAD = 100352
NQ = 4096
PAD_VAL = 1e4
BIGF = 3e38

SC_NC = 2
SC_NS = 16
SC_NW = SC_NC * SC_NS
SC_CHUNK = 64


def _dist_kernel(q_ref, k_ref, dist_ref, bm_ref):
    ki = pl.program_id(0)
    qi = pl.program_id(1)
    rows = pl.ds(qi * TQD, TQD)

    @pl.when(ki == 0)
    def _init():
        bm_ref[rows, :] = jnp.full((TQD, 128), BIGF, jnp.float32)

    q = q_ref[...]
    k = k_ref[...]
    q2 = jnp.sum(q * q, axis=1, keepdims=True)
    k2 = jnp.sum(k * k, axis=1)
    dots = jax.lax.dot_general(
        q, k, (((1,), (1,)), ((), ())),
        preferred_element_type=jnp.float32)
    dist = q2 + k2[None, :] - 2.0 * dots
    dist_ref[...] = dist
    m = jnp.min(dist, axis=1, keepdims=True)
    lane = jax.lax.broadcasted_iota(jnp.int32, (TQD, 128), 1)
    bm_ref[rows, :] = jnp.where(lane == ki, m, bm_ref[rows, :])


def _select_kernel(bm_ref, cb_ref, fi_ref):
    qi = pl.program_id(0)
    lane = jax.lax.broadcasted_iota(jnp.int32, (TQ, 128), 1)
    lane_f = lane.astype(jnp.float32)
    cv = jnp.where(lane < NB, bm_ref[...], BIGF)
    cols = []
    for _ in range(N_NEIGHBOURS):
        m = jnp.min(cv, axis=1, keepdims=True)
        key = jnp.where(cv == m, lane_f, BIGF)
        pos = jnp.min(key, axis=1, keepdims=True)
        cv = jnp.where(key == pos, BIGF, cv)
        cols.append(pos)
    for a, b in ((0, 1), (3, 4), (2, 4), (2, 3), (1, 4),
                 (0, 3), (0, 2), (1, 3), (1, 2)):
        lo = jnp.minimum(cols[a], cols[b])
        hi = jnp.maximum(cols[a], cols[b])
        cols[a], cols[b] = lo, hi
    cb = jnp.concatenate(cols, axis=1).astype(jnp.int32)
    cb_ref[...] = cb
    row = qi * TQ + jax.lax.broadcasted_iota(jnp.int32, (TQ, 1), 0)
    fi_ref[...] = cb * NQ + row


def _sc_gather(table_ref, idx_ref, out_ref, idx_v, rows_v, sem):
    b_per_w = (NQ * N_NEIGHBOURS) // SC_NW
    wid = lax.axis_index("s") * SC_NC + lax.axis_index("c")
    base = wid * b_per_w
    for c in range(b_per_w // SC_CHUNK):
        off = base + c * SC_CHUNK
        pltpu.sync_copy(idx_ref.at[pl.ds(off, SC_CHUNK)], idx_v)
        pltpu.async_copy(table_ref.at[idx_v], rows_v, sem).wait()
        pltpu.sync_copy(rows_v, out_ref.at[pl.ds(off, SC_CHUNK)])


def _final_kernel(g0, g1, g2, g3, g4, cb_ref, scores_ref, idx_ref):
    g = [r[...] for r in (g0, g1, g2, g3, g4)]
    cbf = cb_ref[...].astype(jnp.float32)
    lane_f = jax.lax.broadcasted_iota(
        jnp.int32, (TQ, BK), 1).astype(jnp.float32)
    gpos = [cbf[:, j:j + 1] * BK + lane_f for j in range(N_NEIGHBOURS)]
    vals = []
    idxs = []
    for _ in range(N_NEIGHBOURS):
        s = g[0]
        for j in range(1, N_NEIGHBOURS):
            s = jnp.minimum(s, g[j])
        m = jnp.min(s, axis=1, keepdims=True)
        keys = [jnp.where(g[j] == m, gpos[j], BIGF) for j in range(N_NEIGHBOURS)]
        ks = keys[0]
        for j in range(1, N_NEIGHBOURS):
            ks = jnp.minimum(ks, keys[j])
        pos = jnp.min(ks, axis=1, keepdims=True)
        g = [jnp.where(keys[j] == pos, BIGF, g[j]) for j in range(N_NEIGHBOURS)]
        vals.append(m)
        idxs.append(pos)
    vals5 = jnp.concatenate(vals, axis=1)
    scores_ref[...] = jnp.mean(vals5, axis=1, keepdims=True)
    idx_ref[...] = jnp.concatenate(idxs, axis=1).astype(jnp.int32)


def _gather_candidates(dist_flat, fi_flat):
    nrows = fi_flat.shape[0]
    gather = pl.kernel(
        _sc_gather,
        out_type=jax.ShapeDtypeStruct((nrows, BK), jnp.float32),
        mesh=plsc.VectorSubcoreMesh(core_axis_name="c", subcore_axis_name="s"),
        scratch_types=[
            pltpu.VMEM((SC_CHUNK,), jnp.int32),
            pltpu.VMEM((SC_CHUNK, BK), jnp.float32),
            pltpu.SemaphoreType.DMA,
        ],
    )
    return gather(dist_flat, fi_flat)


@jax.jit
def kernel(queries, keys):
    Q, D = queries.shape
    K, _ = keys.shape
    keys_p = jnp.pad(keys, ((0, K_PAD - K), (0, 0)), constant_values=PAD_VAL)

    dist, bm = pl.pallas_call(
        _dist_kernel,
        grid=(NB, Q // TQD),
        in_specs=[
            pl.BlockSpec((TQD, D), lambda ki, qi: (qi, 0)),
            pl.BlockSpec((BK, D), lambda ki, qi: (ki, 0)),
        ],
        out_specs=[
            pl.BlockSpec((TQD, BK), lambda ki, qi: (ki * (NQ // TQD) + qi, 0)),
            pl.BlockSpec((NQ, 128), lambda ki, qi: (0, 0)),
        ],
        out_shape=[
            jax.ShapeDtypeStruct((NB * Q, BK), jnp.float32),
            jax.ShapeDtypeStruct((Q, 128), jnp.float32),
        ],
        compiler_params=pltpu.CompilerParams(
            dimension_semantics=("arbitrary", "arbitrary"),
        ),
    )(queries, keys_p)

    cb, fi = pl.pallas_call(
        _select_kernel,
        grid=(Q // TQ,),
        in_specs=[pl.BlockSpec((TQ, 128), lambda qi: (qi, 0))],
        out_specs=[
            pl.BlockSpec((TQ, N_NEIGHBOURS), lambda qi: (qi, 0)),
            pl.BlockSpec((TQ, N_NEIGHBOURS), lambda qi: (qi, 0)),
        ],
        out_shape=[
            jax.ShapeDtypeStruct((Q, N_NEIGHBOURS), jnp.int32),
            jax.ShapeDtypeStruct((Q, N_NEIGHBOURS), jnp.int32),
        ],
    )(bm)

    gathered = _gather_candidates(dist, fi.T.reshape(Q * N_NEIGHBOURS))

    scores2d, topk_idx = pl.pallas_call(
        _final_kernel,
        grid=(Q // TQ,),
        in_specs=[
            pl.BlockSpec((TQ, BK), lambda qi, j=j: (j * (NQ // TQ) + qi, 0))
            for j in range(N_NEIGHBOURS)
        ] + [
            pl.BlockSpec((TQ, N_NEIGHBOURS), lambda qi: (qi, 0)),
        ],
        out_specs=[
            pl.BlockSpec((TQ, 1), lambda qi: (qi, 0)),
            pl.BlockSpec((TQ, N_NEIGHBOURS), lambda qi: (qi, 0)),
        ],
        out_shape=[
            jax.ShapeDtypeStruct((Q, 1), jnp.float32),
            jax.ShapeDtypeStruct((Q, N_NEIGHBOURS), jnp.int32),
        ],
    )(gathered, gathered, gathered, gathered, gathered, cb)
    return scores2d[:, 0], topk_idx


# speedup vs baseline: 1.8968x; 1.2506x over previous
import functools

import jax
import jax.numpy as jnp
from jax import lax
from jax.experimental import pallas as pl
from jax.experimental.pallas import tpu as pltpu
from jax.experimental.pallas import tpu_sc as plsc

N_NEIGHBOURS = 5
TQ = 256
TQD = 1024
BK = 1024
NB = 98
K_PAD = 100352
NQ = 4096
PAD_VAL = 1e4
BIGF = 3e38

SC_NC = 2
SC_NS = 16
SC_NW = SC_NC * SC_NS
SC_CHUNK = 64


def _dist_kernel(q_ref, k_ref, dist_ref, bm_ref):
    ki = pl.program_id(0)
    qi = pl.program_id(1)
    rows = pl.ds(qi * TQD, TQD)

    @pl.when(ki == 0)
    def _init():
        bm_ref[rows, :] = jnp.full((TQD, 128), BIGF, jnp.float32)

    q = q_ref[...]
    k = k_ref[...]
    q2 = jnp.sum(q * q, axis=1, keepdims=True)
    k2 = jnp.sum(k * k, axis=1)
    dots = jax.lax.dot_general(
        q, k, (((1,), (1,)), ((), ())),
        preferred_element_type=jnp.float32)
    dist = q2 + k2[None, :] - 2.0 * dots
    dist_ref[...] = dist
    m = jnp.min(dist, axis=1, keepdims=True)
    lane = jax.lax.broadcasted_iota(jnp.int32, (TQD, 128), 1)
    bm_ref[rows, :] = jnp.where(lane == ki, m, bm_ref[rows, :])


def _select_kernel(bm_ref, cb_ref, fi_ref):
    qi = pl.program_id(0)
    lane = jax.lax.broadcasted_iota(jnp.int32, (TQ, 128), 1)
    lane_f = lane.astype(jnp.float32)
    cv = jnp.where(lane < NB, bm_ref[...], BIGF)
    cols = []
    for _ in range(N_NEIGHBOURS):
        m = jnp.min(cv, axis=1, keepdims=True)
        key = jnp.where(cv == m, lane_f, BIGF)
        pos = jnp.min(key, axis=1, keepdims=True)
        cv = jnp.where(key == pos, BIGF, cv)
        cols.append(pos)
    for a, b in ((0, 1), (3, 4), (2, 4), (2, 3), (1, 4),
                 (0, 3), (0, 2), (1, 3), (1, 2)):
        lo = jnp.minimum(cols[a], cols[b])
        hi = jnp.maximum(cols[a], cols[b])
        cols[a], cols[b] = lo, hi
    cb = jnp.concatenate(cols, axis=1).astype(jnp.int32)
    cb_ref[...] = cb
    row = qi * TQ + jax.lax.broadcasted_iota(jnp.int32, (TQ, 1), 0)
    fi_ref[...] = cb * NQ + row


def _sc_gather(table_ref, idx_ref, out_ref, idx_v, rows_v, sem):
    b_per_w = (NQ * N_NEIGHBOURS) // SC_NW
    wid = lax.axis_index("s") * SC_NC + lax.axis_index("c")
    base = wid * b_per_w
    for c in range(b_per_w // SC_CHUNK):
        off = base + c * SC_CHUNK
        pltpu.sync_copy(idx_ref.at[pl.ds(off, SC_CHUNK)], idx_v)
        pltpu.async_copy(table_ref.at[idx_v], rows_v, sem).wait()
        pltpu.sync_copy(rows_v, out_ref.at[pl.ds(off, SC_CHUNK)])


def _final_kernel(g0, g1, g2, g3, g4, cb_ref, scores_ref, idx_ref):
    g = [r[...] for r in (g0, g1, g2, g3, g4)]
    cbf = cb_ref[...].astype(jnp.float32)
    lane_f = jax.lax.broadcasted_iota(
        jnp.int32, (TQ, BK), 1).astype(jnp.float32)
    gpos = [cbf[:, j:j + 1] * BK + lane_f for j in range(N_NEIGHBOURS)]
    vals = []
    idxs = []
    for _ in range(N_NEIGHBOURS):
        s = g[0]
        for j in range(1, N_NEIGHBOURS):
            s = jnp.minimum(s, g[j])
        m = jnp.min(s, axis=1, keepdims=True)
        keys = [jnp.where(g[j] == m, gpos[j], BIGF) for j in range(N_NEIGHBOURS)]
        ks = keys[0]
        for j in range(1, N_NEIGHBOURS):
            ks = jnp.minimum(ks, keys[j])
        pos = jnp.min(ks, axis=1, keepdims=True)
        g = [jnp.where(keys[j] == pos, BIGF, g[j]) for j in range(N_NEIGHBOURS)]
        vals.append(m)
        idxs.append(pos)
    vals5 = jnp.concatenate(vals, axis=1)
    scores_ref[...] = jnp.mean(vals5, axis=1, keepdims=True)
    idx_ref[...] = jnp.concatenate(idxs, axis=1).astype(jnp.int32)


def _gather_candidates(dist_flat, fi_flat):
    nrows = fi_flat.shape[0]
    gather = pl.kernel(
        _sc_gather,
        out_type=jax.ShapeDtypeStruct((nrows, BK), jnp.float32),
        mesh=plsc.VectorSubcoreMesh(core_axis_name="c", subcore_axis_name="s"),
        scratch_types=[
            pltpu.VMEM((SC_CHUNK,), jnp.int32),
            pltpu.VMEM((SC_CHUNK, BK), jnp.float32),
            pltpu.SemaphoreType.DMA,
        ],
    )
    return gather(dist_flat, fi_flat)


@jax.jit
def kernel(queries, keys):
    Q, D = queries.shape
    K, _ = keys.shape
    keys_p = jnp.pad(keys, ((0, K_PAD - K), (0, 0)), constant_values=PAD_VAL)

    dist, bm = pl.pallas_call(
        _dist_kernel,
        grid=(NB, Q // TQD),
        in_specs=[
            pl.BlockSpec((TQD, D), lambda ki, qi: (qi, 0)),
            pl.BlockSpec((BK, D), lambda ki, qi: (ki, 0)),
        ],
        out_specs=[
            pl.BlockSpec((TQD, BK), lambda ki, qi: (ki * (NQ // TQD) + qi, 0)),
            pl.BlockSpec((NQ, 128), lambda ki, qi: (0, 0)),
        ],
        out_shape=[
            jax.ShapeDtypeStruct((NB * Q, BK), jnp.float32),
            jax.ShapeDtypeStruct((Q, 128), jnp.float32),
        ],
        compiler_params=pltpu.CompilerParams(
            dimension_semantics=("arbitrary", "arbitrary"),
        ),
    )(queries, keys_p)

    cb, fi = pl.pallas_call(
        _select_kernel,
        grid=(Q // TQ,),
        in_specs=[pl.BlockSpec((TQ, 128), lambda qi: (qi, 0))],
        out_specs=[
            pl.BlockSpec((TQ, N_NEIGHBOURS), lambda qi: (qi, 0)),
            pl.BlockSpec((TQ, N_NEIGHBOURS), lambda qi: (qi, 0)),
        ],
        out_shape=[
            jax.ShapeDtypeStruct((Q, N_NEIGHBOURS), jnp.int32),
            jax.ShapeDtypeStruct((Q, N_NEIGHBOURS), jnp.int32),
        ],
    )(bm)

    gathered = _gather_candidates(dist, fi.T.reshape(Q * N_NEIGHBOURS))

    scores2d, topk_idx = pl.pallas_call(
        _final_kernel,
        grid=(Q // TQ,),
        in_specs=[
            pl.BlockSpec((TQ, BK), lambda qi, j=j: (j * (NQ // TQ) + qi, 0))
            for j in range(N_NEIGHBOURS)
        ] + [
            pl.BlockSpec((TQ, N_NEIGHBOURS), lambda qi: (qi, 0)),
        ],
        out_specs=[
            pl.BlockSpec((TQ, 1), lambda qi: (qi, 0)),
            pl.BlockSpec((TQ, N_NEIGHBOURS), lambda qi: (qi, 0)),
        ],
        out_shape=[
            jax.ShapeDtypeStruct((Q, 1), jnp.float32),
            jax.ShapeDtypeStruct((Q, N_NEIGHBOURS), jnp.int32),
        ],
    )(gathered, gathered, gathered, gathered, gathered, cb)
    return scores2d[:, 0], topk_idx


# speedup vs baseline: 2.1236x; 1.1196x over previous
import functools

import jax
import jax.numpy as jnp
from jax import lax
from jax.experimental import pallas as pl
from jax.experimental.pallas import tpu as pltpu
from jax.experimental.pallas import tpu_sc as plsc

N_NEIGHBOURS = 5
TQ = 256
TQD = 2048
BK = 1024
NB = 98
K_PAD = 100352
NQ = 4096
PAD_VAL = 1e4
BIGF = 3e38

SC_NC = 2
SC_NS = 16
SC_NW = SC_NC * SC_NS
SC_CHUNK = 64


def _dist_kernel(q_ref, k_ref, dist_ref, bm_ref):
    ki = pl.program_id(0)
    qi = pl.program_id(1)
    rows = pl.ds(qi * TQD, TQD)

    @pl.when(ki == 0)
    def _init():
        bm_ref[rows, :] = jnp.full((TQD, 128), BIGF, jnp.float32)

    q = q_ref[...]
    k = k_ref[...]
    q2 = jnp.sum(q * q, axis=1, keepdims=True)
    k2 = jnp.sum(k * k, axis=1)
    dots = jax.lax.dot_general(
        q, k, (((1,), (1,)), ((), ())),
        preferred_element_type=jnp.float32)
    dist = q2 + k2[None, :] - 2.0 * dots
    dist_ref[...] = dist
    m = jnp.min(dist, axis=1, keepdims=True)
    lane = jax.lax.broadcasted_iota(jnp.int32, (TQD, 128), 1)
    bm_ref[rows, :] = jnp.where(lane == ki, m, bm_ref[rows, :])


def _select_kernel(bm_ref, cb_ref, fi_ref):
    qi = pl.program_id(0)
    lane = jax.lax.broadcasted_iota(jnp.int32, (TQ, 128), 1)
    lane_f = lane.astype(jnp.float32)
    cv = jnp.where(lane < NB, bm_ref[...], BIGF)
    cols = []
    for _ in range(N_NEIGHBOURS):
        m = jnp.min(cv, axis=1, keepdims=True)
        key = jnp.where(cv == m, lane_f, BIGF)
        pos = jnp.min(key, axis=1, keepdims=True)
        cv = jnp.where(key == pos, BIGF, cv)
        cols.append(pos)
    for a, b in ((0, 1), (3, 4), (2, 4), (2, 3), (1, 4),
                 (0, 3), (0, 2), (1, 3), (1, 2)):
        lo = jnp.minimum(cols[a], cols[b])
        hi = jnp.maximum(cols[a], cols[b])
        cols[a], cols[b] = lo, hi
    cb = jnp.concatenate(cols, axis=1).astype(jnp.int32)
    cb_ref[...] = cb
    row = qi * TQ + jax.lax.broadcasted_iota(jnp.int32, (TQ, 1), 0)
    fi_ref[...] = cb * NQ + row


def _sc_gather(table_ref, idx_ref, out_ref, idx_v, rows_v, sem):
    b_per_w = (NQ * N_NEIGHBOURS) // SC_NW
    wid = lax.axis_index("s") * SC_NC + lax.axis_index("c")
    base = wid * b_per_w
    for c in range(b_per_w // SC_CHUNK):
        off = base + c * SC_CHUNK
        pltpu.sync_copy(idx_ref.at[pl.ds(off, SC_CHUNK)], idx_v)
        pltpu.async_copy(table_ref.at[idx_v], rows_v, sem).wait()
        pltpu.sync_copy(rows_v, out_ref.at[pl.ds(off, SC_CHUNK)])


def _final_kernel(g0, g1, g2, g3, g4, cb_ref, scores_ref, idx_ref):
    g = [r[...] for r in (g0, g1, g2, g3, g4)]
    cbf = cb_ref[...].astype(jnp.float32)
    lane_f = jax.lax.broadcasted_iota(
        jnp.int32, (TQ, BK), 1).astype(jnp.float32)
    gpos = [cbf[:, j:j + 1] * BK + lane_f for j in range(N_NEIGHBOURS)]
    vals = []
    idxs = []
    for _ in range(N_NEIGHBOURS):
        s = g[0]
        for j in range(1, N_NEIGHBOURS):
            s = jnp.minimum(s, g[j])
        m = jnp.min(s, axis=1, keepdims=True)
        keys = [jnp.where(g[j] == m, gpos[j], BIGF) for j in range(N_NEIGHBOURS)]
        ks = keys[0]
        for j in range(1, N_NEIGHBOURS):
            ks = jnp.minimum(ks, keys[j])
        pos = jnp.min(ks, axis=1, keepdims=True)
        g = [jnp.where(keys[j] == pos, BIGF, g[j]) for j in range(N_NEIGHBOURS)]
        vals.append(m)
        idxs.append(pos)
    vals5 = jnp.concatenate(vals, axis=1)
    scores_ref[...] = jnp.mean(vals5, axis=1, keepdims=True)
    idx_ref[...] = jnp.concatenate(idxs, axis=1).astype(jnp.int32)


def _gather_candidates(dist_flat, fi_flat):
    nrows = fi_flat.shape[0]
    gather = pl.kernel(
        _sc_gather,
        out_type=jax.ShapeDtypeStruct((nrows, BK), jnp.float32),
        mesh=plsc.VectorSubcoreMesh(core_axis_name="c", subcore_axis_name="s"),
        scratch_types=[
            pltpu.VMEM((SC_CHUNK,), jnp.int32),
            pltpu.VMEM((SC_CHUNK, BK), jnp.float32),
            pltpu.SemaphoreType.DMA,
        ],
    )
    return gather(dist_flat, fi_flat)


@jax.jit
def kernel(queries, keys):
    Q, D = queries.shape
    K, _ = keys.shape
    keys_p = jnp.pad(keys, ((0, K_PAD - K), (0, 0)), constant_values=PAD_VAL)

    dist, bm = pl.pallas_call(
        _dist_kernel,
        grid=(NB, Q // TQD),
        in_specs=[
            pl.BlockSpec((TQD, D), lambda ki, qi: (qi, 0)),
            pl.BlockSpec((BK, D), lambda ki, qi: (ki, 0)),
        ],
        out_specs=[
            pl.BlockSpec((TQD, BK), lambda ki, qi: (ki * (NQ // TQD) + qi, 0)),
            pl.BlockSpec((NQ, 128), lambda ki, qi: (0, 0)),
        ],
        out_shape=[
            jax.ShapeDtypeStruct((NB * Q, BK), jnp.float32),
            jax.ShapeDtypeStruct((Q, 128), jnp.float32),
        ],
        compiler_params=pltpu.CompilerParams(
            dimension_semantics=("arbitrary", "arbitrary"),
        ),
    )(queries, keys_p)

    cb, fi = pl.pallas_call(
        _select_kernel,
        grid=(Q // TQ,),
        in_specs=[pl.BlockSpec((TQ, 128), lambda qi: (qi, 0))],
        out_specs=[
            pl.BlockSpec((TQ, N_NEIGHBOURS), lambda qi: (qi, 0)),
            pl.BlockSpec((TQ, N_NEIGHBOURS), lambda qi: (qi, 0)),
        ],
        out_shape=[
            jax.ShapeDtypeStruct((Q, N_NEIGHBOURS), jnp.int32),
            jax.ShapeDtypeStruct((Q, N_NEIGHBOURS), jnp.int32),
        ],
    )(bm)

    gathered = _gather_candidates(dist, fi.T.reshape(Q * N_NEIGHBOURS))

    scores2d, topk_idx = pl.pallas_call(
        _final_kernel,
        grid=(Q // TQ,),
        in_specs=[
            pl.BlockSpec((TQ, BK), lambda qi, j=j: (j * (NQ // TQ) + qi, 0))
            for j in range(N_NEIGHBOURS)
        ] + [
            pl.BlockSpec((TQ, N_NEIGHBOURS), lambda qi: (qi, 0)),
        ],
        out_specs=[
            pl.BlockSpec((TQ, 1), lambda qi: (qi, 0)),
            pl.BlockSpec((TQ, N_NEIGHBOURS), lambda qi: (qi, 0)),
        ],
        out_shape=[
            jax.ShapeDtypeStruct((Q, 1), jnp.float32),
            jax.ShapeDtypeStruct((Q, N_NEIGHBOURS), jnp.int32),
        ],
    )(gathered, gathered, gathered, gathered, gathered, cb)
    return scores2d[:, 0], topk_idx


# speedup vs baseline: 2.3010x; 1.0835x over previous
import functools

import jax
import jax.numpy as jnp
from jax import lax
from jax.experimental import pallas as pl
from jax.experimental.pallas import tpu as pltpu
from jax.experimental.pallas import tpu_sc as plsc

N_NEIGHBOURS = 5
TQ = 256
TQD = 4096
BK = 1024
NB = 98
K_PAD = 100352
NQ = 4096
PAD_VAL = 1e4
BIGF = 3e38

SC_NC = 2
SC_NS = 16
SC_NW = SC_NC * SC_NS
SC_CHUNK = 64


def _dist_kernel(q_ref, k_ref, dist_ref, bm_ref):
    ki = pl.program_id(0)
    qi = pl.program_id(1)
    rows = pl.ds(qi * TQD, TQD)

    @pl.when(ki == 0)
    def _init():
        bm_ref[rows, :] = jnp.full((TQD, 128), BIGF, jnp.float32)

    q = q_ref[...]
    k = k_ref[...]
    q2 = jnp.sum(q * q, axis=1, keepdims=True)
    k2 = jnp.sum(k * k, axis=1)
    dots = jax.lax.dot_general(
        q, k, (((1,), (1,)), ((), ())),
        preferred_element_type=jnp.float32)
    dist = q2 + k2[None, :] - 2.0 * dots
    dist_ref[...] = dist
    m = jnp.min(dist, axis=1, keepdims=True)
    lane = jax.lax.broadcasted_iota(jnp.int32, (TQD, 128), 1)
    bm_ref[rows, :] = jnp.where(lane == ki, m, bm_ref[rows, :])


def _select_kernel(bm_ref, cb_ref, fi_ref):
    qi = pl.program_id(0)
    lane = jax.lax.broadcasted_iota(jnp.int32, (TQ, 128), 1)
    lane_f = lane.astype(jnp.float32)
    cv = jnp.where(lane < NB, bm_ref[...], BIGF)
    cols = []
    for _ in range(N_NEIGHBOURS):
        m = jnp.min(cv, axis=1, keepdims=True)
        key = jnp.where(cv == m, lane_f, BIGF)
        pos = jnp.min(key, axis=1, keepdims=True)
        cv = jnp.where(key == pos, BIGF, cv)
        cols.append(pos)
    for a, b in ((0, 1), (3, 4), (2, 4), (2, 3), (1, 4),
                 (0, 3), (0, 2), (1, 3), (1, 2)):
        lo = jnp.minimum(cols[a], cols[b])
        hi = jnp.maximum(cols[a], cols[b])
        cols[a], cols[b] = lo, hi
    cb = jnp.concatenate(cols, axis=1).astype(jnp.int32)
    cb_ref[...] = cb
    row = qi * TQ + jax.lax.broadcasted_iota(jnp.int32, (TQ, 1), 0)
    fi_ref[...] = cb * NQ + row


def _sc_gather(table_ref, idx_ref, out_ref, idx_v, rows_v, sem):
    b_per_w = (NQ * N_NEIGHBOURS) // SC_NW
    wid = lax.axis_index("s") * SC_NC + lax.axis_index("c")
    base = wid * b_per_w
    for c in range(b_per_w // SC_CHUNK):
        off = base + c * SC_CHUNK
        pltpu.sync_copy(idx_ref.at[pl.ds(off, SC_CHUNK)], idx_v)
        pltpu.async_copy(table_ref.at[idx_v], rows_v, sem).wait()
        pltpu.sync_copy(rows_v, out_ref.at[pl.ds(off, SC_CHUNK)])


def _final_kernel(g0, g1, g2, g3, g4, cb_ref, scores_ref, idx_ref):
    g = [r[...] for r in (g0, g1, g2, g3, g4)]
    cbf = cb_ref[...].astype(jnp.float32)
    lane_f = jax.lax.broadcasted_iota(
        jnp.int32, (TQ, BK), 1).astype(jnp.float32)
    gpos = [cbf[:, j:j + 1] * BK + lane_f for j in range(N_NEIGHBOURS)]
    vals = []
    idxs = []
    for _ in range(N_NEIGHBOURS):
        s = g[0]
        for j in range(1, N_NEIGHBOURS):
            s = jnp.minimum(s, g[j])
        m = jnp.min(s, axis=1, keepdims=True)
        keys = [jnp.where(g[j] == m, gpos[j], BIGF) for j in range(N_NEIGHBOURS)]
        ks = keys[0]
        for j in range(1, N_NEIGHBOURS):
            ks = jnp.minimum(ks, keys[j])
        pos = jnp.min(ks, axis=1, keepdims=True)
        g = [jnp.where(keys[j] == pos, BIGF, g[j]) for j in range(N_NEIGHBOURS)]
        vals.append(m)
        idxs.append(pos)
    vals5 = jnp.concatenate(vals, axis=1)
    scores_ref[...] = jnp.mean(vals5, axis=1, keepdims=True)
    idx_ref[...] = jnp.concatenate(idxs, axis=1).astype(jnp.int32)


def _gather_candidates(dist_flat, fi_flat):
    nrows = fi_flat.shape[0]
    gather = pl.kernel(
        _sc_gather,
        out_type=jax.ShapeDtypeStruct((nrows, BK), jnp.float32),
        mesh=plsc.VectorSubcoreMesh(core_axis_name="c", subcore_axis_name="s"),
        scratch_types=[
            pltpu.VMEM((SC_CHUNK,), jnp.int32),
            pltpu.VMEM((SC_CHUNK, BK), jnp.float32),
            pltpu.SemaphoreType.DMA,
        ],
    )
    return gather(dist_flat, fi_flat)


@jax.jit
def kernel(queries, keys):
    Q, D = queries.shape
    K, _ = keys.shape
    keys_p = jnp.pad(keys, ((0, K_PAD - K), (0, 0)), constant_values=PAD_VAL)

    dist, bm = pl.pallas_call(
        _dist_kernel,
        grid=(NB, Q // TQD),
        in_specs=[
            pl.BlockSpec((TQD, D), lambda ki, qi: (qi, 0)),
            pl.BlockSpec((BK, D), lambda ki, qi: (ki, 0)),
        ],
        out_specs=[
            pl.BlockSpec((TQD, BK), lambda ki, qi: (ki * (NQ // TQD) + qi, 0)),
            pl.BlockSpec((NQ, 128), lambda ki, qi: (0, 0)),
        ],
        out_shape=[
            jax.ShapeDtypeStruct((NB * Q, BK), jnp.float32),
            jax.ShapeDtypeStruct((Q, 128), jnp.float32),
        ],
        compiler_params=pltpu.CompilerParams(
            dimension_semantics=("arbitrary", "arbitrary"),
        ),
    )(queries, keys_p)

    cb, fi = pl.pallas_call(
        _select_kernel,
        grid=(Q // TQ,),
        in_specs=[pl.BlockSpec((TQ, 128), lambda qi: (qi, 0))],
        out_specs=[
            pl.BlockSpec((TQ, N_NEIGHBOURS), lambda qi: (qi, 0)),
            pl.BlockSpec((TQ, N_NEIGHBOURS), lambda qi: (qi, 0)),
        ],
        out_shape=[
            jax.ShapeDtypeStruct((Q, N_NEIGHBOURS), jnp.int32),
            jax.ShapeDtypeStruct((Q, N_NEIGHBOURS), jnp.int32),
        ],
    )(bm)

    gathered = _gather_candidates(dist, fi.T.reshape(Q * N_NEIGHBOURS))

    scores2d, topk_idx = pl.pallas_call(
        _final_kernel,
        grid=(Q // TQ,),
        in_specs=[
            pl.BlockSpec((TQ, BK), lambda qi, j=j: (j * (NQ // TQ) + qi, 0))
            for j in range(N_NEIGHBOURS)
        ] + [
            pl.BlockSpec((TQ, N_NEIGHBOURS), lambda qi: (qi, 0)),
        ],
        out_specs=[
            pl.BlockSpec((TQ, 1), lambda qi: (qi, 0)),
            pl.BlockSpec((TQ, N_NEIGHBOURS), lambda qi: (qi, 0)),
        ],
        out_shape=[
            jax.ShapeDtypeStruct((Q, 1), jnp.float32),
            jax.ShapeDtypeStruct((Q, N_NEIGHBOURS), jnp.int32),
        ],
    )(gathered, gathered, gathered, gathered, gathered, cb)
    return scores2d[:, 0], topk_idx


# speedup vs baseline: 2.3174x; 1.0071x over previous
import jax
import jax.numpy as jnp
from jax import lax
from jax.experimental import pallas as pl
from jax.experimental.pallas import tpu as pltpu
from jax.experimental.pallas import tpu_sc as plsc

N_NEIGHBOURS = 5
TQ = 256
TQD = 4096
BK = 1024
NB = 98
K_PAD = 100352
NQ = 4096
PAD_VAL = 1e4
BIGF = 3e38

SC_NC = 2
SC_NS = 16
SC_NW = SC_NC * SC_NS
SC_CHUNK = 64


def _dist_kernel(q_ref, k_ref, dist_ref, bm_ref):
    ki = pl.program_id(0)
    qi = pl.program_id(1)
    rows = pl.ds(qi * TQD, TQD)

    @pl.when(ki == 0)
    def _init():
        bm_ref[rows, :] = jnp.full((TQD, 128), BIGF, jnp.float32)

    q = q_ref[...]
    k = k_ref[...]
    q2 = jnp.sum(q * q, axis=1, keepdims=True)
    k2 = jnp.sum(k * k, axis=1)
    dots = jax.lax.dot_general(
        q, k, (((1,), (1,)), ((), ())),
        preferred_element_type=jnp.float32)
    dist = q2 + k2[None, :] - 2.0 * dots
    dist_ref[...] = dist
    m = jnp.min(dist, axis=1, keepdims=True)
    lane = jax.lax.broadcasted_iota(jnp.int32, (TQD, 128), 1)
    bm_ref[rows, :] = jnp.where(lane == ki, m, bm_ref[rows, :])


def _select_kernel(bm_ref, cb_ref, fi_ref):
    qi = pl.program_id(0)
    lane = jax.lax.broadcasted_iota(jnp.int32, (TQ, 128), 1)
    lane_f = lane.astype(jnp.float32)
    cv = jnp.where(lane < NB, bm_ref[...], BIGF)
    cols = []
    for _ in range(N_NEIGHBOURS):
        m = jnp.min(cv, axis=1, keepdims=True)
        key = jnp.where(cv == m, lane_f, BIGF)
        pos = jnp.min(key, axis=1, keepdims=True)
        cv = jnp.where(key == pos, BIGF, cv)
        cols.append(pos)
    for a, b in ((0, 1), (3, 4), (2, 4), (2, 3), (1, 4),
                 (0, 3), (0, 2), (1, 3), (1, 2)):
        lo = jnp.minimum(cols[a], cols[b])
        hi = jnp.maximum(cols[a], cols[b])
        cols[a], cols[b] = lo, hi
    cb = jnp.concatenate(cols, axis=1).astype(jnp.int32)
    cb_ref[...] = cb
    row = qi * TQ + jax.lax.broadcasted_iota(jnp.int32, (TQ, 1), 0)
    fi_ref[...] = cb * NQ + row


def _sc_gather(table_ref, idx_ref, out_ref, idx_v, rows_v, sem):
    b_per_w = (NQ * N_NEIGHBOURS) // SC_NW
    wid = lax.axis_index("s") * SC_NC + lax.axis_index("c")
    base = wid * b_per_w
    for c in range(b_per_w // SC_CHUNK):
        off = base + c * SC_CHUNK
        pltpu.sync_copy(idx_ref.at[pl.ds(off, SC_CHUNK)], idx_v)
        pltpu.async_copy(table_ref.at[idx_v], rows_v, sem).wait()
        pltpu.sync_copy(rows_v, out_ref.at[pl.ds(off, SC_CHUNK)])


def _final_kernel(g0, g1, g2, g3, g4, cb_ref, scores_ref, idx_ref):
    g = [r[...] for r in (g0, g1, g2, g3, g4)]
    cbf = cb_ref[...].astype(jnp.float32)
    lane_f = jax.lax.broadcasted_iota(
        jnp.int32, (TQ, BK), 1).astype(jnp.float32)
    gpos = [cbf[:, j:j + 1] * BK + lane_f for j in range(N_NEIGHBOURS)]
    vals = []
    idxs = []
    for _ in range(N_NEIGHBOURS):
        s = g[0]
        for j in range(1, N_NEIGHBOURS):
            s = jnp.minimum(s, g[j])
        m = jnp.min(s, axis=1, keepdims=True)
        keys = [jnp.where(g[j] == m, gpos[j], BIGF) for j in range(N_NEIGHBOURS)]
        ks = keys[0]
        for j in range(1, N_NEIGHBOURS):
            ks = jnp.minimum(ks, keys[j])
        pos = jnp.min(ks, axis=1, keepdims=True)
        g = [jnp.where(keys[j] == pos, BIGF, g[j]) for j in range(N_NEIGHBOURS)]
        vals.append(m)
        idxs.append(pos)
    vals5 = jnp.concatenate(vals, axis=1)
    scores_ref[...] = jnp.mean(vals5, axis=1, keepdims=True)
    idx_ref[...] = jnp.concatenate(idxs, axis=1).astype(jnp.int32)


def _gather_candidates(dist_flat, fi_flat):
    nrows = fi_flat.shape[0]
    gather = pl.kernel(
        _sc_gather,
        out_type=jax.ShapeDtypeStruct((nrows, BK), jnp.float32),
        mesh=plsc.VectorSubcoreMesh(core_axis_name="c", subcore_axis_name="s"),
        scratch_types=[
            pltpu.VMEM((SC_CHUNK,), jnp.int32),
            pltpu.VMEM((SC_CHUNK, BK), jnp.float32),
            pltpu.SemaphoreType.DMA,
        ],
    )
    return gather(dist_flat, fi_flat)


@jax.jit
def kernel(queries, keys):
    Q, D = queries.shape
    K, _ = keys.shape
    keys_p = jnp.pad(keys, ((0, K_PAD - K), (0, 0)), constant_values=PAD_VAL)

    dist, bm = pl.pallas_call(
        _dist_kernel,
        grid=(NB, Q // TQD),
        in_specs=[
            pl.BlockSpec((TQD, D), lambda ki, qi: (qi, 0)),
            pl.BlockSpec((BK, D), lambda ki, qi: (ki, 0)),
        ],
        out_specs=[
            pl.BlockSpec((TQD, BK), lambda ki, qi: (ki * (NQ // TQD) + qi, 0)),
            pl.BlockSpec((NQ, 128), lambda ki, qi: (0, 0)),
        ],
        out_shape=[
            jax.ShapeDtypeStruct((NB * Q, BK), jnp.float32),
            jax.ShapeDtypeStruct((Q, 128), jnp.float32),
        ],
        compiler_params=pltpu.CompilerParams(
            dimension_semantics=("arbitrary", "arbitrary"),
        ),
    )(queries, keys_p)

    cb, fi = pl.pallas_call(
        _select_kernel,
        grid=(Q // TQ,),
        in_specs=[pl.BlockSpec((TQ, 128), lambda qi: (qi, 0))],
        out_specs=[
            pl.BlockSpec((TQ, N_NEIGHBOURS), lambda qi: (qi, 0)),
            pl.BlockSpec((TQ, N_NEIGHBOURS), lambda qi: (qi, 0)),
        ],
        out_shape=[
            jax.ShapeDtypeStruct((Q, N_NEIGHBOURS), jnp.int32),
            jax.ShapeDtypeStruct((Q, N_NEIGHBOURS), jnp.int32),
        ],
    )(bm)

    gathered = _gather_candidates(dist, fi.T.reshape(Q * N_NEIGHBOURS))

    scores2d, topk_idx = pl.pallas_call(
        _final_kernel,
        grid=(Q // TQ,),
        in_specs=[
            pl.BlockSpec((TQ, BK), lambda qi, j=j: (j * (NQ // TQ) + qi, 0))
            for j in range(N_NEIGHBOURS)
        ] + [
            pl.BlockSpec((TQ, N_NEIGHBOURS), lambda qi: (qi, 0)),
        ],
        out_specs=[
            pl.BlockSpec((TQ, 1), lambda qi: (qi, 0)),
            pl.BlockSpec((TQ, N_NEIGHBOURS), lambda qi: (qi, 0)),
        ],
        out_shape=[
            jax.ShapeDtypeStruct((Q, 1), jnp.float32),
            jax.ShapeDtypeStruct((Q, N_NEIGHBOURS), jnp.int32),
        ],
    )(gathered, gathered, gathered, gathered, gathered, cb)
    return scores2d[:, 0], topk_idx
